# Initial kernel scaffold; baseline (speedup 1.0000x reference)
import jax
import jax.numpy as jnp
from jax.experimental import pallas as pl


def kernel(input, orig_pcd, W_kv, key_gamma, key_beta, val_gamma, val_beta, R):
    raise NotImplementedError("write your pallas kernel here")


# speedup vs baseline: 3.4702x; 3.4702x over previous
import functools

import jax
import jax.numpy as jnp
from jax import lax
from jax.experimental import pallas as pl
from jax.experimental.pallas import tpu as pltpu, tpu_sc as plsc

B = 4
N = 16384
MD = 128
H = 4
F = 32
T = 32
T3 = T * T * T
KC = H * 3
VC = H * F
CP = 144
EPS = 1e-5
NPAIR = B * H
NS = 16
NCORE = 2
PTS_PER_TILE = N // NS
CHUNK = 128
NCHUNK = PTS_PER_TILE // CHUNK


CH1 = 2048
NB1 = N // CH1


def _stats_body(x_ref, w_ref, s1_ref, s2_ref):
    bi = pl.program_id(0)
    ni = pl.program_id(1)

    @pl.when(jnp.logical_and(bi == 0, ni == 0))
    def _():
        s1_ref[...] = jnp.zeros_like(s1_ref)
        s2_ref[...] = jnp.zeros_like(s2_ref)

    x = x_ref[0]
    kv = lax.dot_general(
        w_ref[...], x, (((1,), (0,)), ((), ())),
        preferred_element_type=jnp.float32)
    s1_ref[...] += jnp.sum(kv, axis=1, keepdims=True)
    s2_ref[...] += jnp.sum(kv * kv, axis=1, keepdims=True)


def _stats_call(x, wp):
    return pl.pallas_call(
        _stats_body,
        grid=(B, NB1),
        in_specs=[
            pl.BlockSpec((1, MD, CH1), lambda b, n: (b, 0, n)),
            pl.BlockSpec((CP, MD), lambda b, n: (0, 0)),
        ],
        out_specs=[
            pl.BlockSpec((CP, 1), lambda b, n: (0, 0)),
            pl.BlockSpec((CP, 1), lambda b, n: (0, 0)),
        ],
        out_shape=[
            jax.ShapeDtypeStruct((CP, 1), jnp.float32),
            jax.ShapeDtypeStruct((CP, 1), jnp.float32),
        ],
    )(x, wp)


CH2 = 512
NB2 = N // CH2


def _dense_body(x_ref, o_ref, w_ref, mu_ref, sc_ref, be_ref, rb_ref,
                v_ref, wt_ref, ix_ref, ks_ref, k2_ref):
    bi = pl.program_id(0)
    ni = pl.program_id(1)

    @pl.when(jnp.logical_and(bi == 0, ni == 0))
    def _():
        ks_ref[0, 0] = 0.0
        k2_ref[0, 0] = 0.0

    x = x_ref[0]
    kv = lax.dot_general(
        w_ref[...], x, (((1,), (0,)), ((), ())),
        preferred_element_type=jnp.float32)
    kvn = (kv - mu_ref[...]) * sc_ref[...] + be_ref[...]

    o = o_ref[0]
    otile = jnp.concatenate([o[0:3, :]] * 4 + [o[3:7, :]], axis=0)
    coords16 = kvn[0:16, :] + otile
    keys16 = lax.dot_general(
        rb_ref[...], coords16, (((1,), (0,)), ((), ())),
        preferred_element_type=jnp.float32)
    ks_acc = jnp.sum(keys16[0:KC, :])
    k2_acc = jnp.sum(keys16[0:KC, :] * keys16[0:KC, :])
    w_rows = []
    i_rows = []
    for h in range(H):
        keys = keys16[3 * h:3 * h + 3, :]
        lat = jnp.tanh(keys)
        p = (lat + 1.0) * 0.5 * (T - 1)
        f = jnp.clip(jnp.floor(p), 0.0, T - 2)
        local = p - f
        fi = f.astype(jnp.int32)
        lx, ly, lz = local[0:1, :], local[1:2, :], local[2:3, :]
        base = (fi[0:1, :] * T + fi[1:2, :]) * T + fi[2:3, :]
        wch = []
        ich = []
        for ci in (0, 1):
            wx = lx if ci else 1.0 - lx
            for cj in (0, 1):
                wy = ly if cj else 1.0 - ly
                for ck in (0, 1):
                    wz = lz if ck else 1.0 - lz
                    wch.append(wx * wy * wz)
                    ich.append(base + (ci * T * T + cj * T + ck))
        w_rows.append(jnp.concatenate(wch, axis=0))
        i_rows.append(jnp.concatenate(ich, axis=0))

    ks_ref[0, 0] += ks_acc
    k2_ref[0, 0] += k2_acc
    wt_ref[0] = jnp.concatenate(w_rows, axis=0)
    ix_ref[0] = jnp.concatenate(i_rows, axis=0)
    v_ref[0] = kvn[KC:KC + VC, :]


def _dense_call(x, opad, wp, mu, scale, beta, rflat):
    return pl.pallas_call(
        _dense_body,
        grid=(B, NB2),
        in_specs=[
            pl.BlockSpec((1, MD, CH2), lambda b, n: (b, 0, n)),
            pl.BlockSpec((1, 8, CH2), lambda b, n: (b, 0, n)),
            pl.BlockSpec((CP, MD), lambda b, n: (0, 0)),
            pl.BlockSpec((CP, 1), lambda b, n: (0, 0)),
            pl.BlockSpec((CP, 1), lambda b, n: (0, 0)),
            pl.BlockSpec((CP, 1), lambda b, n: (0, 0)),
            pl.BlockSpec((16, 16), lambda b, n: (0, 0)),
        ],
        out_specs=[
            pl.BlockSpec((1, VC, CH2), lambda b, n: (b, 0, n)),
            pl.BlockSpec((1, 32, CH2), lambda b, n: (b, 0, n)),
            pl.BlockSpec((1, 32, CH2), lambda b, n: (b, 0, n)),
            pl.BlockSpec(memory_space=pltpu.SMEM),
            pl.BlockSpec(memory_space=pltpu.SMEM),
        ],
        out_shape=[
            jax.ShapeDtypeStruct((B, VC, N), jnp.float32),
            jax.ShapeDtypeStruct((B, 32, N), jnp.float32),
            jax.ShapeDtypeStruct((B, 32, N), jnp.int32),
            jax.ShapeDtypeStruct((1, 1), jnp.float32),
            jax.ShapeDtypeStruct((1, 1), jnp.float32),
        ],
    )(x, opad, wp, mu, scale, beta, rflat)


def _splat_body(v_hbm, w_hbm, i_hbm, z_hbm, z_sh, zbuf, vbuf, wbuf, ibuf,
                staged):
    cid = lax.axis_index("c")
    sid = lax.axis_index("s")
    zv = jnp.zeros((16,), jnp.float32)
    for i in range(CHUNK):
        zbuf[i, pl.ds(0, 16)] = zv
        zbuf[i, pl.ds(16, 16)] = zv
    iota = lax.iota(jnp.int32, 16)

    def round_body(r, carry):
        pair = r * NCORE + cid

        def zero_one(i, c):
            pltpu.sync_copy(zbuf, z_sh.at[pl.ds((sid * 16 + i) * CHUNK,
                                                CHUNK)])
            return c

        lax.fori_loop(0, 16, zero_one, 0)
        plsc.subcore_barrier()

        def chunk_body(k, c):
            p0 = sid * PTS_PER_TILE + k * CHUNK
            pltpu.sync_copy(v_hbm.at[pair, :, pl.ds(p0, CHUNK)], vbuf)
            pltpu.sync_copy(w_hbm.at[pair, :, pl.ds(p0, CHUNK)], wbuf)
            pltpu.sync_copy(i_hbm.at[pair, :, pl.ds(p0, CHUNK)], ibuf)

            def grp_body(g, c2):
                goff = g * 16
                rows = [iota + (goff + cc * CHUNK) for cc in range(8)]
                wv = [wbuf[cc, pl.ds(goff, 16)] for cc in range(8)]
                for f in range(F):
                    vv = vbuf[f, pl.ds(goff, 16)]
                    col = jnp.full((16,), f, jnp.int32)
                    for cc in range(8):
                        plsc.store_scatter(staged, [rows[cc], col],
                                           vv * wv[cc])
                return c2

            lax.fori_loop(0, CHUNK // 16, grp_body, 0)
            for cc in range(8):
                pltpu.sync_copy(staged.at[pl.ds(cc * CHUNK, CHUNK)],
                                z_sh.at[ibuf.at[cc]], add=True)
            return c

        lax.fori_loop(0, NCHUNK, chunk_body, 0)
        plsc.subcore_barrier()
        pltpu.sync_copy(z_sh.at[pl.ds(sid * (T3 // NS), T3 // NS)],
                        z_hbm.at[pair, pl.ds(sid * (T3 // NS), T3 // NS)])
        plsc.subcore_barrier()
        return carry

    lax.fori_loop(0, NPAIR // NCORE, round_body, 0)


def _splat_call(v16, w16, i16):
    mesh = plsc.VectorSubcoreMesh(core_axis_name="c", subcore_axis_name="s")
    run = pl.kernel(
        _splat_body,
        out_type=jax.ShapeDtypeStruct((NPAIR, T3, F), jnp.float32),
        mesh=mesh,
        compiler_params=pltpu.CompilerParams(needs_layout_passes=False,
                                             use_tc_tiling_on_sc=False),
        scratch_types=[
            pltpu.VMEM_SHARED((T3, F), jnp.float32),
            pltpu.VMEM((CHUNK, F), jnp.float32),
            pltpu.VMEM((F, CHUNK), jnp.float32),
            pltpu.VMEM((8, CHUNK), jnp.float32),
            pltpu.VMEM((8, CHUNK), jnp.int32),
            pltpu.VMEM((8 * CHUNK, F), jnp.float32),
        ],
    )
    return run(v16, w16, i16)


CB = 2048
NB4 = T3 // CB


def _final_body(z_ref, zt_ref, nnz_ref):
    pi = pl.program_id(0)
    si = pl.program_id(1)

    @pl.when(jnp.logical_and(pi == 0, si == 0))
    def _():
        nnz_ref[0, 0] = 0

    zb = z_ref[0]
    zt_ref[0] = zb.T
    nnz_ref[0, 0] += jnp.sum((jnp.abs(zb) > 1e-9).astype(jnp.int32))


def _final_call(zrows):
    return pl.pallas_call(
        _final_body,
        grid=(NPAIR, NB4),
        in_specs=[pl.BlockSpec((1, CB, F), lambda p, s: (p, s, 0))],
        out_specs=[
            pl.BlockSpec((1, F, CB), lambda p, s: (p, 0, s)),
            pl.BlockSpec(memory_space=pltpu.SMEM),
        ],
        out_shape=[
            jax.ShapeDtypeStruct((NPAIR, F, T3), jnp.float32),
            jax.ShapeDtypeStruct((1, 1), jnp.int32),
        ],
    )(zrows)


def kernel(input, orig_pcd, W_kv, key_gamma, key_beta, val_gamma, val_beta,
           R):
    x = input.astype(jnp.float32)
    wp = jnp.zeros((CP, MD), jnp.float32).at[:KC + VC].set(W_kv)
    s1, s2 = _stats_call(x, wp)
    m = jnp.float32(B * N)
    mean_c = s1[:, 0] / m
    var_c = s2[:, 0] / m - mean_c * mean_c
    gamma = jnp.concatenate(
        [key_gamma, val_gamma, jnp.ones((CP - KC - VC,), jnp.float32)])
    betav = jnp.concatenate(
        [key_beta, val_beta, jnp.zeros((CP - KC - VC,), jnp.float32)])
    scale = gamma / jnp.sqrt(var_c + EPS)
    opad = jnp.zeros((B, 8, N), jnp.float32).at[:, :3].set(orig_pcd)
    rbig = jnp.zeros((16, 16), jnp.float32)
    for h in range(H):
        rbig = rbig.at[3 * h:3 * h + 3, 3 * h:3 * h + 3].set(R[h])

    vout, wout, ixout, ksum, k2sum = _dense_call(
        x, opad, wp, mean_c[:, None], scale[:, None], betav[:, None], rbig)

    v16 = vout.reshape(NPAIR, F, N)
    w16 = wout.reshape(NPAIR, 8, N)
    i16 = ixout.reshape(NPAIR, 8, N)
    zrows = _splat_call(v16, w16, i16)
    zt, nnz = _final_call(zrows)
    z = zt.reshape(B, H, F, T3)
    occ = nnz[0, 0].astype(jnp.float32) / jnp.float32(B * F * H)
    cnt = B * KC * N
    mean_k = ksum[0, 0] / jnp.float32(cnt)
    var_k = (k2sum[0, 0] - ksum[0, 0] * ksum[0, 0] / jnp.float32(cnt)) \
        / jnp.float32(cnt - 1)
    return z, occ, mean_k, var_k
